# baseline (speedup 1.0000x reference)
import functools

import jax
import jax.numpy as jnp
from jax import lax
from jax.experimental import pallas as pl
from jax.experimental.pallas import tpu as pltpu
from jax.experimental.pallas import tpu_sc as plsc

_INFO = plsc.get_sparse_core_info()
_NC, _NS, _L = _INFO.num_cores, _INFO.num_subcores, _INFO.num_lanes
_NW = _NC * _NS


def _make_lookup(NB, CTX, D):
    B = NB * CTX
    BPW = B // _NW
    WPB = _NW // NB
    K = 16
    RPG = 256 // (D // 2)
    mesh = plsc.VectorSubcoreMesh(core_axis_name="c", subcore_axis_name="s")

    @functools.partial(
        pl.kernel,
        mesh=mesh,
        compiler_params=pltpu.CompilerParams(
            use_tc_tiling_on_sc=True, needs_layout_passes=False),
        out_type=jax.ShapeDtypeStruct((NB, CTX, D), jnp.float32),
        scratch_types=[
            pltpu.VMEM((BPW,), jnp.int32),
            pltpu.VMEM_SHARED((_NS * BPW, 128), jnp.int32),
            pltpu.VMEM((BPW, 128), jnp.int32),
            pltpu.VMEM((BPW, D), jnp.float32),
            pltpu.SemaphoreType.DMA,
        ],
    )
    def lookup(table_hbm, idx_hbm, out_hbm, idx_v, stage_s, block_v,
               rows_v, sem):
        cid = lax.axis_index("c")
        sid = lax.axis_index("s")
        wid = sid * _NC + cid
        b = wid // WPB
        t0 = (wid % WPB) * BPW
        pltpu.sync_copy(idx_hbm.at[b, pl.ds(t0, BPW)], idx_v)

        def fire_group(g, carry):
            chunk = idx_v[pl.ds(g * K, K)] // RPG
            for j in range(K):
                pltpu.async_copy(
                    table_hbm.at[pl.ds(chunk[j], 1)],
                    stage_s.at[pl.ds(sid * BPW + g * K + j, 1)],
                    sem,
                )
            return carry

        lax.fori_loop(0, BPW // K, fire_group, 0)

        def drain_group(g, carry):
            for j in range(K):
                pltpu.make_async_copy(
                    table_hbm.at[pl.ds(0, 1)],
                    stage_s.at[pl.ds(sid * BPW + g * K + j, 1)],
                    sem,
                ).wait()
            return carry

        lax.fori_loop(0, BPW // K, drain_group, 0)
        pltpu.sync_copy(stage_s.at[pl.ds(sid * BPW, BPW)], block_v)

        iota1 = lax.iota(jnp.int32, _L)
        iota2 = iota1 * 2
        himask = jnp.full((_L,), -65536, jnp.int32)
        W = D // 2

        def cvt_group(g, carry):
            ids = idx_v[pl.ds(g * K, K)]
            keep = (ids != 0).astype(jnp.float32)
            sub = (ids % RPG) * W
            for j in range(K):
                row = g * K + j
                kj = keep[j]
                rowvec = jnp.full((_L,), 0, jnp.int32) + row
                for c in range(W // _L):
                    w = plsc.load_gather(
                        block_v, [rowvec, iota1 + (sub[j] + c * _L)])
                    lo = plsc.bitcast(w << 16, jnp.float32) * kj
                    hi = plsc.bitcast(w & himask, jnp.float32) * kj
                    cols = iota2 + (c * 2 * _L)
                    plsc.store_scatter(rows_v, [rowvec, cols], lo)
                    plsc.store_scatter(rows_v, [rowvec, cols + 1], hi)
            return carry

        lax.fori_loop(0, BPW // K, cvt_group, 0)
        pltpu.sync_copy(rows_v, out_hbm.at[b, pl.ds(t0, BPW)])

    return lookup


def kernel(inputs, embedding_matrix):
    nb, ctx = inputs.shape
    V, D = embedding_matrix.shape
    table16 = embedding_matrix.astype(jnp.bfloat16).reshape(V, D // 2, 2)
    packed = jax.lax.bitcast_convert_type(table16, jnp.int32)
    packed = packed.reshape(V * (D // 2) // 128, 128)
    return _make_lookup(nb, ctx, D)(packed, inputs)


# speedup vs baseline: 3.4046x; 3.4046x over previous
import functools

import jax
import jax.numpy as jnp
from jax import lax
from jax.experimental import pallas as pl
from jax.experimental.pallas import tpu as pltpu
from jax.experimental.pallas import tpu_sc as plsc

_INFO = plsc.get_sparse_core_info()
_NC, _NS, _L = _INFO.num_cores, _INFO.num_subcores, _INFO.num_lanes
_NW = _NC * _NS


def _make_lookup(NB, CTX, D):
    B = NB * CTX
    BPW = B // _NW
    WPB = _NW // NB
    K = 16
    mesh = plsc.VectorSubcoreMesh(core_axis_name="c", subcore_axis_name="s")

    @functools.partial(
        pl.kernel,
        mesh=mesh,
        compiler_params=pltpu.CompilerParams(
            use_tc_tiling_on_sc=False, needs_layout_passes=False),
        out_type=jax.ShapeDtypeStruct((NB, CTX, D), jnp.float32),
        scratch_types=[
            pltpu.VMEM((BPW,), jnp.int32),
            pltpu.VMEM((BPW, D), jnp.float32),
            pltpu.SemaphoreType.DMA,
        ],
    )
    def lookup(table_hbm, idx_hbm, out_hbm, idx_v, rows_v, sem):
        wid = lax.axis_index("s") * _NC + lax.axis_index("c")
        b = wid // WPB
        t0 = (wid % WPB) * BPW
        pltpu.sync_copy(idx_hbm.at[b, pl.ds(t0, BPW)], idx_v)

        IDX_SLICE = 128
        copies = [
            pltpu.async_copy(
                table_hbm.at[idx_v.at[pl.ds(j * IDX_SLICE, IDX_SLICE)]],
                rows_v.at[pl.ds(j * IDX_SLICE, IDX_SLICE)],
                sem,
            )
            for j in range(BPW // IDX_SLICE)
        ]
        for cp in copies:
            cp.wait()

        zeros = jnp.zeros((_L,), jnp.float32)

        def fix_chunk(c, carry):
            ids = idx_v[pl.ds(c * _L, _L)]
            pad = ids == 0
            npad = jnp.sum(pad.astype(jnp.int32))

            @pl.when(npad > 0)
            def _():
                rows = c * _L + lax.iota(jnp.int32, _L)

                def zero_col(col, carry2):
                    plsc.store_scatter(
                        rows_v,
                        [rows, jnp.full((_L,), 0, jnp.int32) + col],
                        zeros,
                        mask=pad,
                    )
                    return carry2

                lax.fori_loop(0, D, zero_col, 0)

            return carry

        lax.fori_loop(0, BPW // _L, fix_chunk, 0)
        pltpu.sync_copy(rows_v, out_hbm.at[b, pl.ds(t0, BPW)])

    return lookup


def kernel(inputs, embedding_matrix):
    nb, ctx = inputs.shape
    _, D = embedding_matrix.shape
    return _make_lookup(nb, ctx, D)(embedding_matrix, inputs)


# speedup vs baseline: 4.9343x; 1.4493x over previous
import functools

import jax
import jax.numpy as jnp
from jax import lax
from jax.experimental import pallas as pl
from jax.experimental.pallas import tpu as pltpu
from jax.experimental.pallas import tpu_sc as plsc

_INFO = plsc.get_sparse_core_info()
_NC, _NS, _L = _INFO.num_cores, _INFO.num_subcores, _INFO.num_lanes
_NW = _NC * _NS


def _make_lookup(NB, CTX, D):
    B = NB * CTX
    BPW = B // _NW
    WPB = _NW // NB
    K = 16
    mesh = plsc.VectorSubcoreMesh(core_axis_name="c", subcore_axis_name="s")

    @functools.partial(
        pl.kernel,
        mesh=mesh,
        compiler_params=pltpu.CompilerParams(
            use_tc_tiling_on_sc=True, needs_layout_passes=False),
        out_type=jax.ShapeDtypeStruct((NB, CTX, D), jnp.float32),
        scratch_types=[
            pltpu.VMEM((BPW,), jnp.int32),
            pltpu.VMEM((BPW, D), jnp.float32),
            pltpu.SemaphoreType.DMA,
        ],
    )
    def lookup(table_hbm, idx_hbm, out_hbm, idx_v, rows_v, sem):
        wid = lax.axis_index("s") * _NC + lax.axis_index("c")
        b = wid // WPB
        t0 = (wid % WPB) * BPW
        pltpu.sync_copy(idx_hbm.at[b, pl.ds(t0, BPW)], idx_v)

        def fire_group(g, carry):
            chunk = idx_v[pl.ds(g * K, K)]
            for j in range(K):
                pltpu.async_copy(
                    table_hbm.at[pl.ds(chunk[j], 1)],
                    rows_v.at[pl.ds(g * K + j, 1)],
                    sem,
                )
            return carry

        lax.fori_loop(0, BPW // K, fire_group, 0)

        def drain_group(g, carry):
            for j in range(K):
                pltpu.make_async_copy(
                    table_hbm.at[pl.ds(0, 1)],
                    rows_v.at[pl.ds(g * K + j, 1)],
                    sem,
                ).wait()
            return carry

        lax.fori_loop(0, BPW // K, drain_group, 0)

        zeros = jnp.zeros((_L,), jnp.float32)

        def fix_chunk(c, carry):
            ids = idx_v[pl.ds(c * _L, _L)]
            pad = ids == 0
            npad = jnp.sum(pad.astype(jnp.int32))

            @pl.when(npad > 0)
            def _():
                rows = c * _L + lax.iota(jnp.int32, _L)

                def zero_col(col, carry2):
                    plsc.store_scatter(
                        rows_v,
                        [rows, jnp.full((_L,), 0, jnp.int32) + col],
                        zeros,
                        mask=pad,
                    )
                    return carry2

                lax.fori_loop(0, D, zero_col, 0)

            return carry

        lax.fori_loop(0, BPW // _L, fix_chunk, 0)
        pltpu.sync_copy(rows_v, out_hbm.at[b, pl.ds(t0, BPW)])

    return lookup


def kernel(inputs, embedding_matrix):
    nb, ctx = inputs.shape
    _, D = embedding_matrix.shape
    return _make_lookup(nb, ctx, D)(embedding_matrix, inputs)
